# initial kernel scaffold (unmeasured)
import jax
import jax.numpy as jnp
from jax import lax
from jax.experimental import pallas as pl
from jax.experimental.pallas import tpu as pltpu


def kernel(
    x,
):
    def body(*refs):
        pass

    out_shape = jax.ShapeDtypeStruct(..., jnp.float32)
    return pl.pallas_call(body, out_shape=out_shape)(...)


# baseline (device time: 115053 ns/iter reference)
import jax
import jax.numpy as jnp
from jax import lax
from jax.experimental import pallas as pl
from jax.experimental.pallas import tpu as pltpu

N_DEV = 8
M, N = 2048, 1024
CHUNK = M // N_DEV
N_HOPS = 2 * (N_DEV - 1)


def kernel(x):
    x16 = x.reshape(M, N).astype(jnp.bfloat16)

    def body(x_ref, out_ref, comm_ref, send_sems, recv_sems):
        my = lax.axis_index("i")
        left = lax.rem(my + N_DEV - 1, N_DEV)
        right = lax.rem(my + 1, N_DEV)

        barrier_sem = pltpu.get_barrier_semaphore()
        for nbr in (left, right):
            pl.semaphore_signal(
                barrier_sem, inc=1,
                device_id=(nbr,), device_id_type=pl.DeviceIdType.MESH,
            )
        pl.semaphore_wait(barrier_sem, 2)

        comm_ref[0, :, :] = x_ref[pl.ds(my * CHUNK, CHUNK), :]

        for h in range(N_HOPS):
            send_slot = h % 2
            recv_slot = (h + 1) % 2
            rdma = pltpu.make_async_remote_copy(
                src_ref=comm_ref.at[send_slot],
                dst_ref=comm_ref.at[recv_slot],
                send_sem=send_sems.at[h],
                recv_sem=recv_sems.at[h],
                device_id=(right,),
                device_id_type=pl.DeviceIdType.MESH,
            )
            rdma.start()
            rdma.wait()

            if h < N_DEV - 1:
                c = lax.rem(my - h - 1 + N_DEV, N_DEV)
                comm_ref[recv_slot, :, :] = (
                    comm_ref[recv_slot, :, :] + x_ref[pl.ds(c * CHUNK, CHUNK), :]
                )
                if h == N_DEV - 2:
                    out_ref[pl.ds(c * CHUNK, CHUNK), :] = (
                        comm_ref[recv_slot, :, :].astype(jnp.float32)
                    )
            else:
                t = h - (N_DEV - 1)
                c = lax.rem(my - t + N_DEV, N_DEV)
                out_ref[pl.ds(c * CHUNK, CHUNK), :] = (
                    comm_ref[recv_slot, :, :].astype(jnp.float32)
                )

    return pl.pallas_call(
        body,
        out_shape=jax.ShapeDtypeStruct((M, N), jnp.float32),
        in_specs=[pl.BlockSpec(memory_space=pltpu.VMEM)],
        out_specs=pl.BlockSpec(memory_space=pltpu.VMEM),
        scratch_shapes=[
            pltpu.VMEM((2, CHUNK, N), jnp.bfloat16),
            pltpu.SemaphoreType.DMA((N_HOPS,)),
            pltpu.SemaphoreType.DMA((N_HOPS,)),
        ],
        compiler_params=pltpu.CompilerParams(collective_id=0),
    )(x16)


# device time: 61577 ns/iter; 1.8684x vs baseline; 1.8684x over previous
import jax
import jax.numpy as jnp
from jax import lax
from jax.experimental import pallas as pl
from jax.experimental.pallas import tpu as pltpu

N_DEV = 8
M, N = 2048, 1024

GROUPS = ((0, 1024, 0), (1024, 1024, 2))
DIM_MASK = (1, 3, 4)


def kernel(x):
    x16 = x.reshape(M, N).astype(jnp.bfloat16)
    G = len(GROUPS)
    max_half = max(r for _, r, _ in GROUPS) // 2

    def body(x_ref, out_ref, acc_ref, stage_ref, send_sems, recv_sems):
        my = lax.axis_index("i")
        bit0 = my & 1
        bit1 = (my >> 1) & 1
        bit2 = (my >> 2) & 1
        bits = (bit0 ^ bit1, bit1, bit2)

        barrier_sem = pltpu.get_barrier_semaphore()
        for mask in DIM_MASK:
            pl.semaphore_signal(
                barrier_sem, inc=1,
                device_id=(my ^ mask,), device_id_type=pl.DeviceIdType.MESH,
            )
        pl.semaphore_wait(barrier_sem, len(DIM_MASK))

        offs = [jnp.int32(s) for (s, _, _) in GROUPS]

        for k in range(3):
            slot = k % 2
            src = x_ref if k == 0 else acc_ref
            rdmas, meta = [], []
            for g, (start, rows, c) in enumerate(GROUPS):
                d = (k + c) % 3
                size = rows >> (k + 1)
                b = bits[d]
                send_off = offs[g] + (1 - b) * size
                keep_off = offs[g] + b * size
                rdma = pltpu.make_async_remote_copy(
                    src_ref=src.at[pl.ds(send_off, size)],
                    dst_ref=stage_ref.at[g, slot, pl.ds(0, size)],
                    send_sem=send_sems.at[g, k],
                    recv_sem=recv_sems.at[g, k],
                    device_id=(my ^ DIM_MASK[d],),
                    device_id_type=pl.DeviceIdType.MESH,
                )
                rdma.start()
                rdmas.append(rdma)
                meta.append((keep_off, size))
                offs[g] = keep_off
            for g in range(G):
                rdmas[g].wait()
                keep_off, size = meta[g]
                acc_ref[pl.ds(keep_off, size), :] = (
                    src[pl.ds(keep_off, size), :]
                    + stage_ref[g, slot, pl.ds(0, size), :]
                )

        for k in range(3):
            rdmas, newoffs = [], []
            for g, (start, rows, c) in enumerate(GROUPS):
                d = (2 - k + c) % 3
                size = (rows >> 3) << k
                b = bits[d]
                cur = offs[g]
                rdma = pltpu.make_async_remote_copy(
                    src_ref=acc_ref.at[pl.ds(cur, size)],
                    dst_ref=acc_ref.at[pl.ds(cur, size)],
                    send_sem=send_sems.at[g, 3 + k],
                    recv_sem=recv_sems.at[g, 3 + k],
                    device_id=(my ^ DIM_MASK[d],),
                    device_id_type=pl.DeviceIdType.MESH,
                )
                rdma.start()
                rdmas.append(rdma)
                newoffs.append(cur - b * size)
            for g in range(G):
                rdmas[g].wait()
                offs[g] = newoffs[g]

        out_ref[:, :] = acc_ref[:, :].astype(jnp.float32)

    return pl.pallas_call(
        body,
        out_shape=jax.ShapeDtypeStruct((M, N), jnp.float32),
        in_specs=[pl.BlockSpec(memory_space=pltpu.VMEM)],
        out_specs=pl.BlockSpec(memory_space=pltpu.VMEM),
        scratch_shapes=[
            pltpu.VMEM((M, N), jnp.bfloat16),
            pltpu.VMEM((G, 2, max_half, N), jnp.bfloat16),
            pltpu.SemaphoreType.DMA((G, 6)),
            pltpu.SemaphoreType.DMA((G, 6)),
        ],
        compiler_params=pltpu.CompilerParams(collective_id=0),
    )(x16)


# device time: 51489 ns/iter; 2.2345x vs baseline; 1.1959x over previous
import jax
import jax.numpy as jnp
from jax import lax
from jax.experimental import pallas as pl
from jax.experimental.pallas import tpu as pltpu

N_DEV = 8
M, N = 2048, 1024

GROUPS = ((0, 768, 0), (768, 640, 1), (1408, 640, 2))
DIM_MASK = (1, 3, 4)


def kernel(x):
    x16 = x.reshape(M, N).astype(jnp.bfloat16)
    G = len(GROUPS)
    max_half = max(r for _, r, _ in GROUPS) // 2

    def body(x_ref, out_ref, acc_ref, stage_ref, send_sems, recv_sems):
        my = lax.axis_index("i")
        bit0 = my & 1
        bit1 = (my >> 1) & 1
        bit2 = (my >> 2) & 1
        bits = (bit0 ^ bit1, bit1, bit2)

        barrier_sem = pltpu.get_barrier_semaphore()
        for mask in DIM_MASK:
            pl.semaphore_signal(
                barrier_sem, inc=1,
                device_id=(my ^ mask,), device_id_type=pl.DeviceIdType.MESH,
            )
        pl.semaphore_wait(barrier_sem, len(DIM_MASK))

        offs = [jnp.int32(s) for (s, _, _) in GROUPS]

        for k in range(3):
            slot = k % 2
            src = x_ref if k == 0 else acc_ref
            rdmas, meta = [], []
            for g, (start, rows, c) in enumerate(GROUPS):
                d = (k + c) % 3
                size = rows >> (k + 1)
                b = bits[d]
                send_off = offs[g] + (1 - b) * size
                keep_off = offs[g] + b * size
                rdma = pltpu.make_async_remote_copy(
                    src_ref=src.at[pl.ds(send_off, size)],
                    dst_ref=stage_ref.at[g, slot, pl.ds(0, size)],
                    send_sem=send_sems.at[g, k],
                    recv_sem=recv_sems.at[g, k],
                    device_id=(my ^ DIM_MASK[d],),
                    device_id_type=pl.DeviceIdType.MESH,
                )
                rdma.start()
                rdmas.append(rdma)
                meta.append((keep_off, size))
                offs[g] = keep_off
            for g in range(G):
                rdmas[g].wait()
                keep_off, size = meta[g]
                acc_ref[pl.ds(keep_off, size), :] = (
                    src[pl.ds(keep_off, size), :]
                    + stage_ref[g, slot, pl.ds(0, size), :]
                )

        for k in range(3):
            rdmas, newoffs = [], []
            for g, (start, rows, c) in enumerate(GROUPS):
                d = (2 - k + c) % 3
                size = (rows >> 3) << k
                b = bits[d]
                cur = offs[g]
                rdma = pltpu.make_async_remote_copy(
                    src_ref=acc_ref.at[pl.ds(cur, size)],
                    dst_ref=acc_ref.at[pl.ds(cur, size)],
                    send_sem=send_sems.at[g, 3 + k],
                    recv_sem=recv_sems.at[g, 3 + k],
                    device_id=(my ^ DIM_MASK[d],),
                    device_id_type=pl.DeviceIdType.MESH,
                )
                rdma.start()
                rdmas.append(rdma)
                newoffs.append(cur - b * size)
            for g in range(G):
                rdmas[g].wait()
                offs[g] = newoffs[g]

        out_ref[:, :] = acc_ref[:, :].astype(jnp.float32)

    return pl.pallas_call(
        body,
        out_shape=jax.ShapeDtypeStruct((M, N), jnp.float32),
        in_specs=[pl.BlockSpec(memory_space=pltpu.VMEM)],
        out_specs=pl.BlockSpec(memory_space=pltpu.VMEM),
        scratch_shapes=[
            pltpu.VMEM((M, N), jnp.bfloat16),
            pltpu.VMEM((G, 2, max_half, N), jnp.bfloat16),
            pltpu.SemaphoreType.DMA((G, 6)),
            pltpu.SemaphoreType.DMA((G, 6)),
        ],
        compiler_params=pltpu.CompilerParams(collective_id=0),
    )(x16)


# device time: 49572 ns/iter; 2.3209x vs baseline; 1.0387x over previous
import jax
import jax.numpy as jnp
from jax import lax
from jax.experimental import pallas as pl
from jax.experimental.pallas import tpu as pltpu

N_DEV = 8
M, N = 2048, 1024

GROUPS = ((0, 768, 0), (768, 640, 1), (1408, 640, 2))
DIM_MASK = (1, 3, 4)


def kernel(x):
    x16 = x.reshape(M, N).astype(jnp.bfloat16)
    G = len(GROUPS)
    max_half = max(r for _, r, _ in GROUPS) // 2

    def body(x_ref, out_ref, stage_ref, send_sems, recv_sems):
        my = lax.axis_index("i")
        bit0 = my & 1
        bit1 = (my >> 1) & 1
        bit2 = (my >> 2) & 1
        bits = (bit0 ^ bit1, bit1, bit2)

        barrier_sem = pltpu.get_barrier_semaphore()
        for mask in DIM_MASK:
            pl.semaphore_signal(
                barrier_sem, inc=1,
                device_id=(my ^ mask,), device_id_type=pl.DeviceIdType.MESH,
            )
        pl.semaphore_wait(barrier_sem, len(DIM_MASK))

        offs = [jnp.int32(s) for (s, _, _) in GROUPS]

        for k in range(3):
            slot = k % 2
            src = x_ref if k == 0 else out_ref
            rdmas, meta = [], []
            for g, (start, rows, c) in enumerate(GROUPS):
                d = (k + c) % 3
                size = rows >> (k + 1)
                b = bits[d]
                send_off = offs[g] + (1 - b) * size
                keep_off = offs[g] + b * size
                rdma = pltpu.make_async_remote_copy(
                    src_ref=src.at[pl.ds(send_off, size)],
                    dst_ref=stage_ref.at[g, slot, pl.ds(0, size)],
                    send_sem=send_sems.at[g, k],
                    recv_sem=recv_sems.at[g, k],
                    device_id=(my ^ DIM_MASK[d],),
                    device_id_type=pl.DeviceIdType.MESH,
                )
                rdma.start()
                rdmas.append(rdma)
                meta.append((keep_off, size))
                offs[g] = keep_off
            for g in range(G):
                rdmas[g].wait()
                keep_off, size = meta[g]
                out_ref[pl.ds(keep_off, size), :] = (
                    src[pl.ds(keep_off, size), :]
                    + stage_ref[g, slot, pl.ds(0, size), :]
                )

        for k in range(3):
            rdmas, newoffs = [], []
            for g, (start, rows, c) in enumerate(GROUPS):
                d = (2 - k + c) % 3
                size = (rows >> 3) << k
                b = bits[d]
                cur = offs[g]
                rdma = pltpu.make_async_remote_copy(
                    src_ref=out_ref.at[pl.ds(cur, size)],
                    dst_ref=out_ref.at[pl.ds(cur, size)],
                    send_sem=send_sems.at[g, 3 + k],
                    recv_sem=recv_sems.at[g, 3 + k],
                    device_id=(my ^ DIM_MASK[d],),
                    device_id_type=pl.DeviceIdType.MESH,
                )
                rdma.start()
                rdmas.append(rdma)
                newoffs.append(cur - b * size)
            for g in range(G):
                rdmas[g].wait()
                offs[g] = newoffs[g]

    return pl.pallas_call(
        body,
        out_shape=jax.ShapeDtypeStruct((M, N), jnp.bfloat16),
        in_specs=[pl.BlockSpec(memory_space=pltpu.VMEM)],
        out_specs=pl.BlockSpec(memory_space=pltpu.VMEM),
        scratch_shapes=[
            pltpu.VMEM((G, 2, max_half, N), jnp.bfloat16),
            pltpu.SemaphoreType.DMA((G, 6)),
            pltpu.SemaphoreType.DMA((G, 6)),
        ],
        compiler_params=pltpu.CompilerParams(collective_id=0),
    )(x16)


# device time: 49526 ns/iter; 2.3231x vs baseline; 1.0009x over previous
import jax
import jax.numpy as jnp
from jax import lax
from jax.experimental import pallas as pl
from jax.experimental.pallas import tpu as pltpu

N_DEV = 8
M, N = 2048, 1024

GROUPS = ((0, 768, 0), (768, 640, 1), (1408, 640, 2))
DIM_MASK = (1, 3, 4)


def kernel(x):
    x32 = x.reshape(M, N)
    G = len(GROUPS)
    max_half = max(r for _, r, _ in GROUPS) // 2
    half_total = sum(r for _, r, _ in GROUPS) // 2

    def body(x_ref, out_ref, stage_ref, send0_ref, send_sems, recv_sems):
        my = lax.axis_index("i")
        bit0 = my & 1
        bit1 = (my >> 1) & 1
        bit2 = (my >> 2) & 1
        bits = (bit0 ^ bit1, bit1, bit2)

        seg0 = []
        so = 0
        for g, (start, rows, c) in enumerate(GROUPS):
            d = c % 3
            size = rows >> 1
            b = bits[d]
            send_off = start + (1 - b) * size
            keep_off = start + b * size
            send0_ref[pl.ds(so, size), :] = (
                x_ref[pl.ds(send_off, size), :].astype(jnp.bfloat16)
            )
            seg0.append((so, size, send_off, keep_off))
            so += size

        barrier_sem = pltpu.get_barrier_semaphore()
        for mask in DIM_MASK:
            pl.semaphore_signal(
                barrier_sem, inc=1,
                device_id=(my ^ mask,), device_id_type=pl.DeviceIdType.MESH,
            )
        pl.semaphore_wait(barrier_sem, len(DIM_MASK))

        offs = [None] * G

        for k in range(3):
            slot = k % 2
            rdmas, meta = [], []
            for g, (start, rows, c) in enumerate(GROUPS):
                d = (k + c) % 3
                size = rows >> (k + 1)
                b = bits[d]
                if k == 0:
                    s0_row, _, _, keep_off = seg0[g]
                    src_slice = send0_ref.at[pl.ds(s0_row, size)]
                else:
                    send_off = offs[g] + (1 - b) * size
                    keep_off = offs[g] + b * size
                    src_slice = out_ref.at[pl.ds(send_off, size)]
                rdma = pltpu.make_async_remote_copy(
                    src_ref=src_slice,
                    dst_ref=stage_ref.at[g, slot, pl.ds(0, size)],
                    send_sem=send_sems.at[g, k],
                    recv_sem=recv_sems.at[g, k],
                    device_id=(my ^ DIM_MASK[d],),
                    device_id_type=pl.DeviceIdType.MESH,
                )
                rdma.start()
                rdmas.append(rdma)
                meta.append((keep_off, size))
                offs[g] = keep_off
            for g in range(G):
                rdmas[g].wait()
                keep_off, size = meta[g]
                if k == 0:
                    local = x_ref[pl.ds(keep_off, size), :].astype(jnp.bfloat16)
                else:
                    local = out_ref[pl.ds(keep_off, size), :]
                out_ref[pl.ds(keep_off, size), :] = (
                    local + stage_ref[g, slot, pl.ds(0, size), :]
                )

        for k in range(3):
            rdmas, newoffs = [], []
            for g, (start, rows, c) in enumerate(GROUPS):
                d = (2 - k + c) % 3
                size = (rows >> 3) << k
                b = bits[d]
                cur = offs[g]
                rdma = pltpu.make_async_remote_copy(
                    src_ref=out_ref.at[pl.ds(cur, size)],
                    dst_ref=out_ref.at[pl.ds(cur, size)],
                    send_sem=send_sems.at[g, 3 + k],
                    recv_sem=recv_sems.at[g, 3 + k],
                    device_id=(my ^ DIM_MASK[d],),
                    device_id_type=pl.DeviceIdType.MESH,
                )
                rdma.start()
                rdmas.append(rdma)
                newoffs.append(cur - b * size)
            for g in range(G):
                rdmas[g].wait()
                offs[g] = newoffs[g]

    return pl.pallas_call(
        body,
        out_shape=jax.ShapeDtypeStruct((M, N), jnp.bfloat16),
        in_specs=[pl.BlockSpec(memory_space=pltpu.VMEM)],
        out_specs=pl.BlockSpec(memory_space=pltpu.VMEM),
        scratch_shapes=[
            pltpu.VMEM((G, 2, max_half, N), jnp.bfloat16),
            pltpu.VMEM((half_total, N), jnp.bfloat16),
            pltpu.SemaphoreType.DMA((G, 6)),
            pltpu.SemaphoreType.DMA((G, 6)),
        ],
        compiler_params=pltpu.CompilerParams(collective_id=0),
    )(x32)


# device time: 41428 ns/iter; 2.7772x vs baseline; 1.1955x over previous
import jax
import jax.numpy as jnp
from jax import lax
from jax.experimental import pallas as pl
from jax.experimental.pallas import tpu as pltpu

N_DEV = 8
M, N = 2048, 1024

GROUPS = ((0, 768, 0), (768, 640, 1), (1408, 640, 2))
DIM_MASK = (1, 3, 4)


def kernel(x):
    x32 = x.reshape(M, N)
    G = len(GROUPS)
    max_half = max(r for _, r, _ in GROUPS) // 2
    half_total = sum(r for _, r, _ in GROUPS) // 2

    def body(x_ref, out_ref, stage_ref, send0_ref, send_sems, recv_sems):
        my = lax.axis_index("i")
        bit0 = my & 1
        bit1 = (my >> 1) & 1
        bit2 = (my >> 2) & 1
        bits = (bit0 ^ bit1, bit1, bit2)

        gm = []
        s0_rows = []
        so = 0
        for g, (start, rows, c) in enumerate(GROUPS):
            d = ((0 + c) % 3, (1 + c) % 3, (2 + c) % 3)
            b = (bits[d[0]], bits[d[1]], bits[d[2]])
            size = (rows >> 1, rows >> 2, rows >> 3)
            keep0 = start + b[0] * size[0]
            keep1 = keep0 + b[1] * size[1]
            keep2 = keep1 + b[2] * size[2]
            gm.append((start, rows, d, b, size, keep0, keep1, keep2))
            s0_rows.append(so)
            so += size[0]

        def xfer(src, dst, g, idx, sub, d):
            r = pltpu.make_async_remote_copy(
                src_ref=src,
                dst_ref=dst,
                send_sem=send_sems.at[g, 2 * idx + sub],
                recv_sem=recv_sems.at[g, 2 * idx + sub],
                device_id=(my ^ DIM_MASK[d],),
                device_id_type=pl.DeviceIdType.MESH,
            )
            r.start()
            return r

        for g, (start, rows, d, b, size, keep0, keep1, keep2) in enumerate(gm):
            send_off = start + (1 - b[0]) * size[0]
            send0_ref[pl.ds(s0_rows[g], size[0]), :] = (
                x_ref[pl.ds(send_off, size[0]), :].astype(jnp.bfloat16)
            )

        barrier_sem = pltpu.get_barrier_semaphore()
        for mask in DIM_MASK:
            pl.semaphore_signal(
                barrier_sem, inc=1,
                device_id=(my ^ mask,), device_id_type=pl.DeviceIdType.MESH,
            )
        pl.semaphore_wait(barrier_sem, len(DIM_MASK))

        r = {}

        for g, (start, rows, d, b, size, keep0, keep1, keep2) in enumerate(gm):
            h0 = size[1]
            for sub in (0, 1):
                off = (1 - b[1]) * h0 if sub == 0 else b[1] * h0
                r[g, "rs0", sub] = xfer(
                    send0_ref.at[pl.ds(s0_rows[g] + off, h0)],
                    stage_ref.at[g, 0, pl.ds(off, h0)],
                    g, 0, sub, d[0],
                )

        for g, (start, rows, d, b, size, keep0, keep1, keep2) in enumerate(gm):
            h0, h1 = size[1], size[2]
            off = (1 - b[1]) * h0
            r[g, "rs0", 0].wait()
            out_ref[pl.ds(keep0 + off, h0), :] = (
                x_ref[pl.ds(keep0 + off, h0), :].astype(jnp.bfloat16)
                + stage_ref[g, 0, pl.ds(off, h0), :]
            )
            send1 = keep0 + (1 - b[1]) * size[1]
            for sub in (0, 1):
                soff = (1 - b[2]) * h1 if sub == 0 else b[2] * h1
                r[g, "rs1", sub] = xfer(
                    out_ref.at[pl.ds(send1 + soff, h1)],
                    stage_ref.at[g, 1, pl.ds(soff, h1)],
                    g, 1, sub, d[1],
                )

        for g, (start, rows, d, b, size, keep0, keep1, keep2) in enumerate(gm):
            h0 = size[1]
            off = b[1] * h0
            r[g, "rs0", 1].wait()
            out_ref[pl.ds(keep0 + off, h0), :] = (
                x_ref[pl.ds(keep0 + off, h0), :].astype(jnp.bfloat16)
                + stage_ref[g, 0, pl.ds(off, h0), :]
            )

        for g, (start, rows, d, b, size, keep0, keep1, keep2) in enumerate(gm):
            h1 = size[2]
            off = (1 - b[2]) * h1
            r[g, "rs1", 0].wait()
            out_ref[pl.ds(keep1 + off, h1), :] = (
                out_ref[pl.ds(keep1 + off, h1), :]
                + stage_ref[g, 1, pl.ds(off, h1), :]
            )
            send2 = keep1 + (1 - b[2]) * size[2]
            r[g, "rs2", 0] = xfer(
                out_ref.at[pl.ds(send2, size[2])],
                stage_ref.at[g, 2, pl.ds(0, size[2])],
                g, 2, 0, d[2],
            )

        for g, (start, rows, d, b, size, keep0, keep1, keep2) in enumerate(gm):
            h1 = size[2]
            off = b[2] * h1
            r[g, "rs1", 1].wait()
            out_ref[pl.ds(keep1 + off, h1), :] = (
                out_ref[pl.ds(keep1 + off, h1), :]
                + stage_ref[g, 1, pl.ds(off, h1), :]
            )

        for g, (start, rows, d, b, size, keep0, keep1, keep2) in enumerate(gm):
            s = size[2]
            r[g, "rs2", 0].wait()
            out_ref[pl.ds(keep2, s), :] = (
                out_ref[pl.ds(keep2, s), :] + stage_ref[g, 2, pl.ds(0, s), :]
            )
            r[g, "ag0", 0] = xfer(
                out_ref.at[pl.ds(keep2, s)], out_ref.at[pl.ds(keep2, s)],
                g, 3, 0, d[2],
            )
            r[g, "ag1", 0] = xfer(
                out_ref.at[pl.ds(keep2, s)], out_ref.at[pl.ds(keep2, s)],
                g, 4, 0, d[1],
            )

        for g, (start, rows, d, b, size, keep0, keep1, keep2) in enumerate(gm):
            s = size[2]
            m1 = keep2 - b[2] * s
            recv0 = m1 + (1 - b[2]) * s
            r[g, "ag0", 0].wait()
            r[g, "ag1", 1] = xfer(
                out_ref.at[pl.ds(recv0, s)], out_ref.at[pl.ds(recv0, s)],
                g, 4, 1, d[1],
            )
            r[g, "ag2", 0] = xfer(
                out_ref.at[pl.ds(m1, 2 * s)], out_ref.at[pl.ds(m1, 2 * s)],
                g, 5, 0, d[0],
            )

        for g, (start, rows, d, b, size, keep0, keep1, keep2) in enumerate(gm):
            s = size[2]
            m1 = keep2 - b[2] * s
            m2 = m1 - b[1] * 2 * s
            recv1 = m2 + (1 - b[1]) * 2 * s
            r[g, "ag1", 0].wait()
            r[g, "ag1", 1].wait()
            r[g, "ag2", 1] = xfer(
                out_ref.at[pl.ds(recv1, 2 * s)], out_ref.at[pl.ds(recv1, 2 * s)],
                g, 5, 1, d[0],
            )

        for g in range(G):
            r[g, "ag2", 0].wait()
            r[g, "ag2", 1].wait()

    return pl.pallas_call(
        body,
        out_shape=jax.ShapeDtypeStruct((M, N), jnp.bfloat16),
        in_specs=[pl.BlockSpec(memory_space=pltpu.VMEM)],
        out_specs=pl.BlockSpec(memory_space=pltpu.VMEM),
        scratch_shapes=[
            pltpu.VMEM((G, 3, max_half, N), jnp.bfloat16),
            pltpu.VMEM((half_total, N), jnp.bfloat16),
            pltpu.SemaphoreType.DMA((G, 12)),
            pltpu.SemaphoreType.DMA((G, 12)),
        ],
        compiler_params=pltpu.CompilerParams(collective_id=0),
    )(x32)
